# baseline (speedup 1.0000x reference)
import functools

import jax
import jax.numpy as jnp
from jax import lax
from jax.experimental import pallas as pl
from jax.experimental.pallas import tpu as pltpu
from jax.experimental.pallas import tpu_sc as plsc

NN = 50000
EE = 800000
HH = 64
HC = 32
NPAD = 51200
EPAD = 819200
EROWS = EPAD // 128
SENT = 50000
NC, NS = 2, 16
TSTRIPE = NPAD // NS

BLK = 3200
GRID = NPAD // BLK

HB = 8
HNB = EROWS // (NC * NS) // HB
AB = 8
ANB = EROWS // NS // AB
RD = 4


def _sc_mesh():
    return plsc.VectorSubcoreMesh(
        core_axis_name="c", subcore_axis_name="s",
        num_cores=NC, num_subcores=NS)


def _hist_body(epad, out, oep, hs_sp, hd_sp, zb, ones_v, ib, *sems):
    cid = lax.axis_index("c")
    sid = lax.axis_index("s")
    wid = sid * NC + cid
    ssem = sems[:4]
    dsem = sems[4:]

    def _z(i, c):
        zb[pl.ds(i * 16, 16)] = jnp.zeros((16,), jnp.float32)
        return c
    lax.fori_loop(0, TSTRIPE // 16, _z, 0)

    def _o(i, c):
        ones_v[pl.ds(i * 16, 16)] = jnp.ones((16,), jnp.float32)
        return c
    lax.fori_loop(0, 128 // 16, _o, 0)

    pltpu.sync_copy(zb, hs_sp.at[pl.ds(sid * TSTRIPE, TSTRIPE)])
    pltpu.sync_copy(zb, hd_sp.at[pl.ds(sid * TSTRIPE, TSTRIPE)])
    plsc.subcore_barrier()

    base = wid * (HNB * HB)

    def _stage(blk, slot):
        sl = pl.ds(base + blk * HB, HB)
        pltpu.sync_copy(epad.at[0, sl], ib.at[0, slot])
        pltpu.sync_copy(epad.at[1, sl], ib.at[1, slot])
        pltpu.sync_copy(ib.at[0, slot], oep.at[0, sl])
        pltpu.sync_copy(ib.at[1, slot], oep.at[1, sl])

    _stage(0, 0)

    def _blk(blk, c):
        slot = lax.rem(blk, 2)
        nslot = 1 - slot
        for r in range(HB):
            if r == 4:
                @pl.when(blk < HNB - 1)
                def _():
                    _stage(blk + 1, nslot)
            b2 = (r + 2) % 4
            if r < 2:
                @pl.when(blk > 0)
                def _():
                    pltpu.make_async_copy(
                        ones_v, hs_sp.at[ib.at[0, 0, r]], ssem[b2]).wait()
                    pltpu.make_async_copy(
                        ones_v, hd_sp.at[ib.at[1, 0, r]], dsem[b2]).wait()
            else:
                pltpu.make_async_copy(
                    ones_v, hs_sp.at[ib.at[0, 0, r]], ssem[b2]).wait()
                pltpu.make_async_copy(
                    ones_v, hd_sp.at[ib.at[1, 0, r]], dsem[b2]).wait()
            pltpu.async_copy(ones_v, hs_sp.at[ib.at[0, slot, r]],
                             ssem[r % 4], add=True)
            pltpu.async_copy(ones_v, hd_sp.at[ib.at[1, slot, r]],
                             dsem[r % 4], add=True)
        return c
    lax.fori_loop(0, HNB, _blk, 0)

    for r in range(2, 4):
        pltpu.make_async_copy(ones_v, hs_sp.at[ib.at[0, 0, r]],
                              ssem[r]).wait()
        pltpu.make_async_copy(ones_v, hd_sp.at[ib.at[1, 0, r]],
                              dsem[r]).wait()

    plsc.subcore_barrier()
    sl = pl.ds(sid * TSTRIPE, TSTRIPE)
    pltpu.sync_copy(hs_sp.at[sl], out.at[cid, 0, sl])
    pltpu.sync_copy(hd_sp.at[sl], out.at[cid, 1, sl])


@functools.cache
def _hist_kernel():
    return pl.kernel(
        _hist_body,
        out_type=(jax.ShapeDtypeStruct((NC, 2, NPAD), jnp.float32),
                  jax.ShapeDtypeStruct((2, EROWS, 128), jnp.int32)),
        mesh=_sc_mesh(),
        compiler_params=pltpu.CompilerParams(use_tc_tiling_on_sc=False),
        scratch_types=[
            pltpu.VMEM_SHARED((NPAD,), jnp.float32),
            pltpu.VMEM_SHARED((NPAD,), jnp.float32),
            pltpu.VMEM((TSTRIPE,), jnp.float32),
            pltpu.VMEM((128,), jnp.float32),
            pltpu.VMEM((2, 2, HB, 128), jnp.int32),
        ] + [pltpu.SemaphoreType.DMA] * 8,
    )


def _hist_call(epad):
    return _hist_kernel()(epad)


def _agg_body(hp, epad, out, acc, zb, rows, sib, dib, *sems):
    cid = lax.axis_index("c")
    sid = lax.axis_index("s")
    gsem = sems[:RD]
    ssem = sems[RD:]

    def _z(i, c):
        zb[i // 2, pl.ds((i % 2) * 16, 16)] = jnp.zeros((16,), jnp.float32)
        return c
    lax.fori_loop(0, 128 * HC // 16, _z, 0)

    def _zc(i, c):
        pltpu.sync_copy(zb, acc.at[pl.ds(sid * TSTRIPE + i * 128, 128)])
        return c
    lax.fori_loop(0, TSTRIPE // 128, _zc, 0)
    plsc.subcore_barrier()

    base = sid * (ANB * AB)

    def _stage(blk, slot):
        pltpu.sync_copy(epad.at[0, pl.ds(base + blk * AB, AB)], sib.at[slot])
        pltpu.sync_copy(epad.at[1, pl.ds(base + blk * AB, AB)], dib.at[slot])

    _stage(0, 0)
    for r in range(2):
        pltpu.async_copy(hp.at[cid].at[sib.at[0, r]], rows.at[r], gsem[r])

    def _blk(blk, c):
        slot = lax.rem(blk, 2)
        nslot = 1 - slot
        for r in range(AB):
            if r == 4:
                @pl.when(blk < ANB - 1)
                def _():
                    _stage(blk + 1, nslot)
            b2 = (r + 2) % RD
            if r < 2:
                @pl.when(blk > 0)
                def _():
                    pltpu.make_async_copy(
                        rows.at[b2], acc.at[dib.at[nslot, r + 6]],
                        ssem[b2]).wait()

                pltpu.async_copy(hp.at[cid].at[sib.at[slot, r + 2]],
                                 rows.at[b2], gsem[b2])
            elif r < 6:
                pltpu.make_async_copy(
                    rows.at[b2], acc.at[dib.at[slot, r - 2]],
                    ssem[b2]).wait()
                pltpu.async_copy(hp.at[cid].at[sib.at[slot, r + 2]],
                                 rows.at[b2], gsem[b2])
            else:
                pltpu.make_async_copy(
                    rows.at[b2], acc.at[dib.at[slot, r - 2]],
                    ssem[b2]).wait()

                @pl.when(blk < ANB - 1)
                def _():
                    pltpu.async_copy(hp.at[cid].at[sib.at[nslot, r - 6]],
                                     rows.at[b2], gsem[b2])
            pltpu.make_async_copy(
                hp.at[cid].at[sib.at[slot, r]], rows.at[r % RD],
                gsem[r % RD]).wait()
            pltpu.async_copy(rows.at[r % RD], acc.at[dib.at[slot, r]],
                             ssem[r % RD], add=True)
        return c
    lax.fori_loop(0, ANB, _blk, 0)

    for r in range(2, RD):
        pltpu.make_async_copy(rows.at[r], acc.at[dib.at[0, r]],
                              ssem[r]).wait()

    plsc.subcore_barrier()
    sl = pl.ds(sid * TSTRIPE, TSTRIPE)
    pltpu.sync_copy(acc.at[sl], out.at[sl, pl.ds(cid * HC, HC)])


@functools.cache
def _agg_kernel():
    return pl.kernel(
        _agg_body,
        out_type=jax.ShapeDtypeStruct((NPAD, 128), jnp.float32),
        mesh=_sc_mesh(),
        compiler_params=pltpu.CompilerParams(use_tc_tiling_on_sc=False),
        scratch_types=[
            pltpu.VMEM_SHARED((NPAD, HC), jnp.float32),
            pltpu.VMEM((128, HC), jnp.float32),
            pltpu.VMEM((RD, 128, HC), jnp.float32),
            pltpu.VMEM((2, AB, 128), jnp.int32),
            pltpu.VMEM((2, AB, 128), jnp.int32),
        ] + [pltpu.SemaphoreType.DMA] * (2 * RD),
    )


def _agg_call(hps, epad):
    return _agg_kernel()(hps, epad)


def _b1_body(h4, cs, cd, mx):
    h = h4[...]
    a = h[0:1] + h[2:3]
    d = h[1:2] + h[3:4]
    cs[...] = a
    cd[...] = d
    ii = lax.broadcasted_iota(jnp.int32, (1, NPAD), 1)
    mx[...] = jnp.reshape(jnp.max(jnp.where(ii < NN, a, -1.0)), (1, 1))


def _b1_call(h4):
    return pl.pallas_call(
        _b1_body,
        out_shape=(
            jax.ShapeDtypeStruct((1, NPAD), jnp.float32),
            jax.ShapeDtypeStruct((1, NPAD), jnp.float32),
            jax.ShapeDtypeStruct((1, 1), jnp.float32),
        ),
    )(h4)


def _b2_body(ni, w1, mx, hh):
    m = jnp.maximum(mx[0, 0], 1.0)
    degf = ni[:, 0:1] / m
    x = jnp.concatenate([ni[:, 2:4], degf], axis=1)
    h = jnp.dot(x, w1[...], preferred_element_type=jnp.float32)
    di = lax.rsqrt(ni[:, 1:2] + 1.0)
    hp = h * di
    hh[...] = jnp.concatenate(
        [hp, di, jnp.zeros((BLK, 127 - HH), jnp.float32)], axis=1)


def _b2_call(ni, w1, mx):
    return pl.pallas_call(
        _b2_body,
        grid=(GRID,),
        in_specs=[
            pl.BlockSpec((BLK, 4), lambda i: (i, 0)),
            pl.BlockSpec((3, HH), lambda i: (0, 0)),
            pl.BlockSpec((1, 1), lambda i: (0, 0)),
        ],
        out_specs=pl.BlockSpec((BLK, 128), lambda i: (i, 0)),
        out_shape=jax.ShapeDtypeStruct((NPAD, 128), jnp.float32),
    )(ni, w1, mx)


def _d_body(agg, hh1, b1, w2, hh2):
    a = agg[:, :HH]
    hp1 = hh1[:, :HH]
    di = hh1[:, HH:HH + 1]
    x1 = jnp.maximum(di * (a + hp1) + b1[...], 0.0)
    h2 = jnp.dot(x1, w2[...], preferred_element_type=jnp.float32)
    hh2[...] = jnp.concatenate(
        [h2 * di, di, jnp.zeros((BLK, 127 - HH), jnp.float32)], axis=1)


def _d_call(agg, hh1, b1, w2):
    return pl.pallas_call(
        _d_body,
        grid=(GRID,),
        in_specs=[
            pl.BlockSpec((BLK, 128), lambda i: (i, 0)),
            pl.BlockSpec((BLK, 128), lambda i: (i, 0)),
            pl.BlockSpec((1, HH), lambda i: (0, 0)),
            pl.BlockSpec((HH, HH), lambda i: (0, 0)),
        ],
        out_specs=pl.BlockSpec((BLK, 128), lambda i: (i, 0)),
        out_shape=jax.ShapeDtypeStruct((NPAD, 128), jnp.float32),
    )(agg, hh1, b1, w2)


def _e1_body(agg, hh2, b2, wst, bst, wmut, bmut, wlst, blst,
             wv1, bv1, wv2, bv2,
             s_o, mu_o, ls_o, macc, sacc, xsum, lse, val):
    i = pl.program_id(0)
    a = agg[:, :HH]
    hp2 = hh2[:, :HH]
    di = hh2[:, HH:HH + 1]
    x2 = jnp.maximum(di * (a + hp2) + b2[...], 0.0)
    cdims = (((1,), (1,)), ((), ()))
    s = lax.dot_general(wst[...], x2, cdims,
                        preferred_element_type=jnp.float32) + bst[...]
    mu_o[...] = lax.dot_general(wmut[...], x2, cdims,
                                preferred_element_type=jnp.float32) + bmut[...]
    ls_o[...] = jnp.clip(
        lax.dot_general(wlst[...], x2, cdims,
                        preferred_element_type=jnp.float32) + blst[...],
        -4.0, 2.0)
    s_o[...] = s

    cidx = i * BLK + lax.broadcasted_iota(jnp.int32, (1, BLK), 1)
    mask = cidx < NN
    sm = jnp.where(mask, s, -1e30)
    bmax = jnp.max(sm)
    ridx = i * BLK + lax.broadcasted_iota(jnp.int32, (BLK, 1), 0)
    xs = jnp.sum(jnp.where(ridx < NN, x2, 0.0), axis=0, keepdims=True)

    bmax11 = jnp.reshape(bmax, (1, 1))

    @pl.when(i == 0)
    def _():
        macc[...] = bmax11
        sacc[...] = jnp.reshape(jnp.sum(jnp.exp(sm - bmax)), (1, 1))
        xsum[...] = xs

    @pl.when(i > 0)
    def _():
        m0 = macc[...]
        mn = jnp.maximum(m0, bmax11)
        sacc[...] = (sacc[...] * jnp.exp(m0 - mn)
                     + jnp.reshape(jnp.sum(jnp.exp(sm - mn)), (1, 1)))
        macc[...] = mn
        xsum[...] = xsum[...] + xs

    @pl.when(i == GRID - 1)
    def _():
        lse[...] = macc[...] + jnp.log(sacc[...])
        ge = xsum[...] * (1.0 / NN)
        v1 = jnp.maximum(
            jnp.dot(ge, wv1[...], preferred_element_type=jnp.float32)
            + bv1[...], 0.0)
        val[...] = jnp.dot(v1, wv2[...],
                           preferred_element_type=jnp.float32) + bv2[...]


def _e1_call(agg, hh2, b2, wst, bst, wmut, bmut, wlst, blst,
             wv1, bv1, wv2, bv2):
    def small(r, c):
        return pl.BlockSpec((r, c), lambda i: (0, 0))
    return pl.pallas_call(
        _e1_body,
        grid=(GRID,),
        in_specs=[
            pl.BlockSpec((BLK, 128), lambda i: (i, 0)),
            pl.BlockSpec((BLK, 128), lambda i: (i, 0)),
            small(1, HH), small(1, HH), small(1, 1),
            small(2, HH), small(2, 1), small(2, HH), small(2, 1),
            small(HH, HH), small(1, HH), small(HH, 1), small(1, 1),
        ],
        out_specs=(
            pl.BlockSpec((1, BLK), lambda i: (0, i)),
            pl.BlockSpec((2, BLK), lambda i: (0, i)),
            pl.BlockSpec((2, BLK), lambda i: (0, i)),
            small(1, 1), small(1, 1), small(1, HH), small(1, 1), small(1, 1),
        ),
        out_shape=(
            jax.ShapeDtypeStruct((1, NPAD), jnp.float32),
            jax.ShapeDtypeStruct((2, NPAD), jnp.float32),
            jax.ShapeDtypeStruct((2, NPAD), jnp.float32),
            jax.ShapeDtypeStruct((1, 1), jnp.float32),
            jax.ShapeDtypeStruct((1, 1), jnp.float32),
            jax.ShapeDtypeStruct((1, HH), jnp.float32),
            jax.ShapeDtypeStruct((1, 1), jnp.float32),
            jax.ShapeDtypeStruct((1, 1), jnp.float32),
        ),
    )(agg, hh2, b2, wst, bst, wmut, bmut, wlst, blst, wv1, bv1, wv2, bv2)


def _e2_body(s, lse, out):
    out[...] = s[...] - lse[...]


def _e2_call(s, lse):
    return pl.pallas_call(
        _e2_body,
        out_shape=jax.ShapeDtypeStruct((1, NPAD), jnp.float32),
    )(s, lse)


def kernel(coords, edge_index, W1, b1, W2, b2, Ws, bs, Wmu, bmu,
           Wls, bls, Wv1, bv1, Wv2, bv2):
    pad_idx = SENT + jnp.arange(EPAD - EE, dtype=jnp.int32) % (NPAD - NN)
    fill = jnp.broadcast_to(pad_idx, (2, EPAD - EE))
    epad = jnp.concatenate(
        [edge_index.astype(jnp.int32), fill], axis=1).reshape(2, EROWS, 128)
    coords_pad = jnp.pad(coords, ((0, NPAD - NN), (0, 0)))

    hist, epad_sc = _hist_call(epad)
    cs_row, cd_row, mx = _b1_call(hist.reshape(4, NPAD))
    nodein = jnp.concatenate(
        [cs_row.reshape(NPAD, 1), cd_row.reshape(NPAD, 1), coords_pad],
        axis=1)

    def _split(hh):
        return jnp.transpose(hh[:, :HH].reshape(NPAD, 2, HC), (1, 0, 2))

    hh1 = _b2_call(nodein, W1, mx)
    hps1 = _split(hh1)
    agg1 = _agg_call(hps1, epad_sc)
    hh2 = _d_call(agg1, hh1, b1.reshape(1, HH), W2)
    hps2 = _split(hh2)
    agg2 = _agg_call(hps2, epad_sc)

    s, mu_t, ls_t, _, _, _, lse, val = _e1_call(
        agg2, hh2, b2.reshape(1, HH), Ws.T, bs.reshape(1, 1),
        Wmu.T, bmu.reshape(2, 1), Wls.T, bls.reshape(2, 1),
        Wv1, bv1.reshape(1, HH), Wv2, bv2.reshape(1, 1))
    logits = _e2_call(s, lse)

    return (logits[0, :NN], mu_t[:, :NN].T, ls_t[:, :NN].T, val[0, 0])


# speedup vs baseline: 1.0438x; 1.0438x over previous
import functools

import jax
import jax.numpy as jnp
from jax import lax
from jax.experimental import pallas as pl
from jax.experimental.pallas import tpu as pltpu
from jax.experimental.pallas import tpu_sc as plsc

NN = 50000
EE = 800000
HH = 64
HC = 32
NPAD = 51200
EPAD = 819200
EROWS = EPAD // 128
SENT = 50000
NC, NS = 2, 16
TSTRIPE = NPAD // NS

BLK = 3200
GRID = NPAD // BLK

HB = 8
HNB = EROWS // (NC * NS) // HB
AB = 8
ANB = EROWS // NS // AB
RD = 4


def _sc_mesh():
    return plsc.VectorSubcoreMesh(
        core_axis_name="c", subcore_axis_name="s",
        num_cores=NC, num_subcores=NS)


def _hist_body(epad, out, oep, hs_sp, hd_sp, zb, ones_v, ib, *sems):
    cid = lax.axis_index("c")
    sid = lax.axis_index("s")
    wid = sid * NC + cid
    ssem = sems[:4]
    dsem = sems[4:]

    def _z(i, c):
        zb[pl.ds(i * 16, 16)] = jnp.zeros((16,), jnp.float32)
        return c
    lax.fori_loop(0, TSTRIPE // 16, _z, 0)

    def _o(i, c):
        ones_v[pl.ds(i * 16, 16)] = jnp.ones((16,), jnp.float32)
        return c
    lax.fori_loop(0, 128 // 16, _o, 0)

    pltpu.sync_copy(zb, hs_sp.at[pl.ds(sid * TSTRIPE, TSTRIPE)])
    pltpu.sync_copy(zb, hd_sp.at[pl.ds(sid * TSTRIPE, TSTRIPE)])
    plsc.subcore_barrier()

    base = wid * (HNB * HB)

    def _stage(blk, slot):
        sl = pl.ds(base + blk * HB, HB)
        pltpu.sync_copy(epad.at[0, sl], ib.at[0, slot])
        pltpu.sync_copy(epad.at[1, sl], ib.at[1, slot])
        pltpu.sync_copy(ib.at[0, slot], oep.at[0, sl])
        pltpu.sync_copy(ib.at[1, slot], oep.at[1, sl])

    _stage(0, 0)

    def _blk(blk, c):
        slot = lax.rem(blk, 2)
        nslot = 1 - slot
        for r in range(HB):
            if r == 4:
                @pl.when(blk < HNB - 1)
                def _():
                    _stage(blk + 1, nslot)
            b2 = (r + 2) % 4
            if r < 2:
                @pl.when(blk > 0)
                def _():
                    pltpu.make_async_copy(
                        ones_v, hs_sp.at[ib.at[0, 0, r]], ssem[b2]).wait()
                    pltpu.make_async_copy(
                        ones_v, hd_sp.at[ib.at[1, 0, r]], dsem[b2]).wait()
            else:
                pltpu.make_async_copy(
                    ones_v, hs_sp.at[ib.at[0, 0, r]], ssem[b2]).wait()
                pltpu.make_async_copy(
                    ones_v, hd_sp.at[ib.at[1, 0, r]], dsem[b2]).wait()
            pltpu.async_copy(ones_v, hs_sp.at[ib.at[0, slot, r]],
                             ssem[r % 4], add=True)
            pltpu.async_copy(ones_v, hd_sp.at[ib.at[1, slot, r]],
                             dsem[r % 4], add=True)
        return c
    lax.fori_loop(0, HNB, _blk, 0)

    for r in range(2, 4):
        pltpu.make_async_copy(ones_v, hs_sp.at[ib.at[0, 0, r]],
                              ssem[r]).wait()
        pltpu.make_async_copy(ones_v, hd_sp.at[ib.at[1, 0, r]],
                              dsem[r]).wait()

    plsc.subcore_barrier()
    sl = pl.ds(sid * TSTRIPE, TSTRIPE)
    pltpu.sync_copy(hs_sp.at[sl], out.at[cid, 0, sl])
    pltpu.sync_copy(hd_sp.at[sl], out.at[cid, 1, sl])


@functools.cache
def _hist_kernel():
    return pl.kernel(
        _hist_body,
        out_type=(jax.ShapeDtypeStruct((NC, 2, NPAD), jnp.float32),
                  jax.ShapeDtypeStruct((2, EROWS, 128), jnp.int32)),
        mesh=_sc_mesh(),
        compiler_params=pltpu.CompilerParams(use_tc_tiling_on_sc=False),
        scratch_types=[
            pltpu.VMEM_SHARED((NPAD,), jnp.float32),
            pltpu.VMEM_SHARED((NPAD,), jnp.float32),
            pltpu.VMEM((TSTRIPE,), jnp.float32),
            pltpu.VMEM((128,), jnp.float32),
            pltpu.VMEM((2, 2, HB, 128), jnp.int32),
        ] + [pltpu.SemaphoreType.DMA] * 8,
    )


def _hist_call(epad):
    return _hist_kernel()(epad)


def _agg_body(hp, epad, out, acc, zb, rows, sib, dib, *sems):
    cid = lax.axis_index("c")
    sid = lax.axis_index("s")
    gsem = sems[:RD]
    ssem = sems[RD:]

    def _z(i, c):
        zb[i // 2, pl.ds((i % 2) * 16, 16)] = jnp.zeros((16,), jnp.float32)
        return c
    lax.fori_loop(0, 128 * HC // 16, _z, 0)

    def _zc(i, c):
        pltpu.sync_copy(zb, acc.at[pl.ds(sid * TSTRIPE + i * 128, 128)])
        return c
    lax.fori_loop(0, TSTRIPE // 128, _zc, 0)
    plsc.subcore_barrier()

    base = sid * (ANB * AB)

    def _stage(blk, slot):
        pltpu.sync_copy(epad.at[0, pl.ds(base + blk * AB, AB)], sib.at[slot])
        pltpu.sync_copy(epad.at[1, pl.ds(base + blk * AB, AB)], dib.at[slot])

    _stage(0, 0)
    for r in range(2):
        pltpu.async_copy(hp.at[cid].at[sib.at[0, r]], rows.at[r], gsem[r])

    def _blk(blk, c):
        slot = lax.rem(blk, 2)
        nslot = 1 - slot
        for r in range(AB):
            if r == 4:
                @pl.when(blk < ANB - 1)
                def _():
                    _stage(blk + 1, nslot)
            b2 = (r + 2) % RD
            if r < 2:
                @pl.when(blk > 0)
                def _():
                    pltpu.make_async_copy(
                        rows.at[b2], acc.at[dib.at[nslot, r + 6]],
                        ssem[b2]).wait()

                pltpu.async_copy(hp.at[cid].at[sib.at[slot, r + 2]],
                                 rows.at[b2], gsem[b2])
            elif r < 6:
                pltpu.make_async_copy(
                    rows.at[b2], acc.at[dib.at[slot, r - 2]],
                    ssem[b2]).wait()
                pltpu.async_copy(hp.at[cid].at[sib.at[slot, r + 2]],
                                 rows.at[b2], gsem[b2])
            else:
                pltpu.make_async_copy(
                    rows.at[b2], acc.at[dib.at[slot, r - 2]],
                    ssem[b2]).wait()

                @pl.when(blk < ANB - 1)
                def _():
                    pltpu.async_copy(hp.at[cid].at[sib.at[nslot, r - 6]],
                                     rows.at[b2], gsem[b2])
            pltpu.make_async_copy(
                hp.at[cid].at[sib.at[slot, r]], rows.at[r % RD],
                gsem[r % RD]).wait()
            pltpu.async_copy(rows.at[r % RD], acc.at[dib.at[slot, r]],
                             ssem[r % RD], add=True)
        return c
    lax.fori_loop(0, ANB, _blk, 0)

    for r in range(2, RD):
        pltpu.make_async_copy(rows.at[r], acc.at[dib.at[0, r]],
                              ssem[r]).wait()

    plsc.subcore_barrier()
    sl = pl.ds(sid * TSTRIPE, TSTRIPE)
    pltpu.sync_copy(acc.at[sl], out.at[sl, pl.ds(cid * HC, HC)])


@functools.cache
def _agg_kernel():
    return pl.kernel(
        _agg_body,
        out_type=jax.ShapeDtypeStruct((NPAD, 128), jnp.float32),
        mesh=_sc_mesh(),
        compiler_params=pltpu.CompilerParams(use_tc_tiling_on_sc=False),
        scratch_types=[
            pltpu.VMEM_SHARED((NPAD, HC), jnp.float32),
            pltpu.VMEM((128, HC), jnp.float32),
            pltpu.VMEM((RD, 128, HC), jnp.float32),
            pltpu.VMEM((2, AB, 128), jnp.int32),
            pltpu.VMEM((2, AB, 128), jnp.int32),
        ] + [pltpu.SemaphoreType.DMA] * (2 * RD),
    )


def _agg_call(hps, epad):
    return _agg_kernel()(hps, epad)


def _b1_body(h4, cs, cd, mx):
    h = h4[...]
    a = h[0:1] + h[2:3]
    d = h[1:2] + h[3:4]
    cs[...] = a
    cd[...] = d
    ii = lax.broadcasted_iota(jnp.int32, (1, NPAD), 1)
    mx[...] = jnp.reshape(jnp.max(jnp.where(ii < NN, a, -1.0)), (1, 1))


def _b1_call(h4):
    return pl.pallas_call(
        _b1_body,
        out_shape=(
            jax.ShapeDtypeStruct((1, NPAD), jnp.float32),
            jax.ShapeDtypeStruct((1, NPAD), jnp.float32),
            jax.ShapeDtypeStruct((1, 1), jnp.float32),
        ),
    )(h4)


def _b2_body(ni, w1, mx, hh):
    m = jnp.maximum(mx[0, 0], 1.0)
    degf = ni[:, 0:1] / m
    x = jnp.concatenate([ni[:, 2:4], degf], axis=1)
    h = jnp.dot(x, w1[...], preferred_element_type=jnp.float32)
    di = lax.rsqrt(ni[:, 1:2] + 1.0)
    hp = h * di
    hh[...] = jnp.concatenate(
        [hp, di, jnp.zeros((BLK, 127 - HH), jnp.float32)], axis=1)


def _b2_body2(ni, w1, mx, hh, hps):
    _b2_body(ni, w1, mx, hh)
    hp = hh[:, :HH]
    hps[...] = jnp.stack([hp[:, :HC], hp[:, HC:]], axis=0)


def _b2_call(ni, w1, mx):
    return pl.pallas_call(
        _b2_body2,
        grid=(GRID,),
        in_specs=[
            pl.BlockSpec((BLK, 4), lambda i: (i, 0)),
            pl.BlockSpec((3, HH), lambda i: (0, 0)),
            pl.BlockSpec((1, 1), lambda i: (0, 0)),
        ],
        out_specs=(pl.BlockSpec((BLK, 128), lambda i: (i, 0)),
                   pl.BlockSpec((2, BLK, HC), lambda i: (0, i, 0))),
        out_shape=(jax.ShapeDtypeStruct((NPAD, 128), jnp.float32),
                   jax.ShapeDtypeStruct((2, NPAD, HC), jnp.float32)),
    )(ni, w1, mx)


def _d_body(agg, hh1, b1, w2, hh2):
    a = agg[:, :HH]
    hp1 = hh1[:, :HH]
    di = hh1[:, HH:HH + 1]
    x1 = jnp.maximum(di * (a + hp1) + b1[...], 0.0)
    h2 = jnp.dot(x1, w2[...], preferred_element_type=jnp.float32)
    hh2[...] = jnp.concatenate(
        [h2 * di, di, jnp.zeros((BLK, 127 - HH), jnp.float32)], axis=1)


def _d_body2(agg, hh1, b1, w2, hh2, hps):
    _d_body(agg, hh1, b1, w2, hh2)
    hp = hh2[:, :HH]
    hps[...] = jnp.stack([hp[:, :HC], hp[:, HC:]], axis=0)


def _d_call(agg, hh1, b1, w2):
    return pl.pallas_call(
        _d_body2,
        grid=(GRID,),
        in_specs=[
            pl.BlockSpec((BLK, 128), lambda i: (i, 0)),
            pl.BlockSpec((BLK, 128), lambda i: (i, 0)),
            pl.BlockSpec((1, HH), lambda i: (0, 0)),
            pl.BlockSpec((HH, HH), lambda i: (0, 0)),
        ],
        out_specs=(pl.BlockSpec((BLK, 128), lambda i: (i, 0)),
                   pl.BlockSpec((2, BLK, HC), lambda i: (0, i, 0))),
        out_shape=(jax.ShapeDtypeStruct((NPAD, 128), jnp.float32),
                   jax.ShapeDtypeStruct((2, NPAD, HC), jnp.float32)),
    )(agg, hh1, b1, w2)


def _e1_body(agg, hh2, b2, wst, bst, wmut, bmut, wlst, blst,
             wv1, bv1, wv2, bv2,
             s_o, mu_o, ls_o, macc, sacc, xsum, lse, val):
    i = pl.program_id(0)
    a = agg[:, :HH]
    hp2 = hh2[:, :HH]
    di = hh2[:, HH:HH + 1]
    x2 = jnp.maximum(di * (a + hp2) + b2[...], 0.0)
    cdims = (((1,), (1,)), ((), ()))
    s = lax.dot_general(wst[...], x2, cdims,
                        preferred_element_type=jnp.float32) + bst[...]
    mu_o[...] = lax.dot_general(wmut[...], x2, cdims,
                                preferred_element_type=jnp.float32) + bmut[...]
    ls_o[...] = jnp.clip(
        lax.dot_general(wlst[...], x2, cdims,
                        preferred_element_type=jnp.float32) + blst[...],
        -4.0, 2.0)
    s_o[...] = s

    cidx = i * BLK + lax.broadcasted_iota(jnp.int32, (1, BLK), 1)
    mask = cidx < NN
    sm = jnp.where(mask, s, -1e30)
    bmax = jnp.max(sm)
    ridx = i * BLK + lax.broadcasted_iota(jnp.int32, (BLK, 1), 0)
    xs = jnp.sum(jnp.where(ridx < NN, x2, 0.0), axis=0, keepdims=True)

    bmax11 = jnp.reshape(bmax, (1, 1))

    @pl.when(i == 0)
    def _():
        macc[...] = bmax11
        sacc[...] = jnp.reshape(jnp.sum(jnp.exp(sm - bmax)), (1, 1))
        xsum[...] = xs

    @pl.when(i > 0)
    def _():
        m0 = macc[...]
        mn = jnp.maximum(m0, bmax11)
        sacc[...] = (sacc[...] * jnp.exp(m0 - mn)
                     + jnp.reshape(jnp.sum(jnp.exp(sm - mn)), (1, 1)))
        macc[...] = mn
        xsum[...] = xsum[...] + xs

    @pl.when(i == GRID - 1)
    def _():
        lse[...] = macc[...] + jnp.log(sacc[...])
        ge = xsum[...] * (1.0 / NN)
        v1 = jnp.maximum(
            jnp.dot(ge, wv1[...], preferred_element_type=jnp.float32)
            + bv1[...], 0.0)
        val[...] = jnp.dot(v1, wv2[...],
                           preferred_element_type=jnp.float32) + bv2[...]


def _e1_call(agg, hh2, b2, wst, bst, wmut, bmut, wlst, blst,
             wv1, bv1, wv2, bv2):
    def small(r, c):
        return pl.BlockSpec((r, c), lambda i: (0, 0))
    return pl.pallas_call(
        _e1_body,
        grid=(GRID,),
        in_specs=[
            pl.BlockSpec((BLK, 128), lambda i: (i, 0)),
            pl.BlockSpec((BLK, 128), lambda i: (i, 0)),
            small(1, HH), small(1, HH), small(1, 1),
            small(2, HH), small(2, 1), small(2, HH), small(2, 1),
            small(HH, HH), small(1, HH), small(HH, 1), small(1, 1),
        ],
        out_specs=(
            pl.BlockSpec((1, BLK), lambda i: (0, i)),
            pl.BlockSpec((2, BLK), lambda i: (0, i)),
            pl.BlockSpec((2, BLK), lambda i: (0, i)),
            small(1, 1), small(1, 1), small(1, HH), small(1, 1), small(1, 1),
        ),
        out_shape=(
            jax.ShapeDtypeStruct((1, NPAD), jnp.float32),
            jax.ShapeDtypeStruct((2, NPAD), jnp.float32),
            jax.ShapeDtypeStruct((2, NPAD), jnp.float32),
            jax.ShapeDtypeStruct((1, 1), jnp.float32),
            jax.ShapeDtypeStruct((1, 1), jnp.float32),
            jax.ShapeDtypeStruct((1, HH), jnp.float32),
            jax.ShapeDtypeStruct((1, 1), jnp.float32),
            jax.ShapeDtypeStruct((1, 1), jnp.float32),
        ),
    )(agg, hh2, b2, wst, bst, wmut, bmut, wlst, blst, wv1, bv1, wv2, bv2)


def _e2_body(s, lse, out):
    out[...] = s[...] - lse[...]


def _e2_call(s, lse):
    return pl.pallas_call(
        _e2_body,
        out_shape=jax.ShapeDtypeStruct((1, NPAD), jnp.float32),
    )(s, lse)


def kernel(coords, edge_index, W1, b1, W2, b2, Ws, bs, Wmu, bmu,
           Wls, bls, Wv1, bv1, Wv2, bv2):
    pad_idx = SENT + jnp.arange(EPAD - EE, dtype=jnp.int32) % (NPAD - NN)
    fill = jnp.broadcast_to(pad_idx, (2, EPAD - EE))
    epad = jnp.concatenate(
        [edge_index.astype(jnp.int32), fill], axis=1).reshape(2, EROWS, 128)
    coords_pad = jnp.pad(coords, ((0, NPAD - NN), (0, 0)))

    hist, epad_sc = _hist_call(epad)
    cs_row, cd_row, mx = _b1_call(hist.reshape(4, NPAD))
    nodein = jnp.concatenate(
        [cs_row.reshape(NPAD, 1), cd_row.reshape(NPAD, 1), coords_pad],
        axis=1)

    hh1, hps1 = _b2_call(nodein, W1, mx)
    agg1 = _agg_call(hps1, epad_sc)
    hh2, hps2 = _d_call(agg1, hh1, b1.reshape(1, HH), W2)
    agg2 = _agg_call(hps2, epad_sc)

    s, mu_t, ls_t, _, _, _, lse, val = _e1_call(
        agg2, hh2, b2.reshape(1, HH), Ws.T, bs.reshape(1, 1),
        Wmu.T, bmu.reshape(2, 1), Wls.T, bls.reshape(2, 1),
        Wv1, bv1.reshape(1, HH), Wv2, bv2.reshape(1, 1))
    logits = _e2_call(s, lse)

    return (logits[0, :NN], mu_t[:, :NN].T, ls_t[:, :NN].T, val[0, 0])
